# Initial kernel scaffold; baseline (speedup 1.0000x reference)
import jax
import jax.numpy as jnp
from jax.experimental import pallas as pl


def kernel(U, edge_index, A_values, W, B, V_w):
    raise NotImplementedError("write your pallas kernel here")


# speedup vs baseline: 6.2931x; 6.2931x over previous
import functools
import jax
import jax.numpy as jnp
from jax import lax
from jax.experimental import pallas as pl
from jax.experimental.pallas import tpu as pltpu
from jax.experimental.pallas import tpu_sc as plsc

N = 10000
E = 160000
NFEAT = 128
NHID = 128
NCLASS = 16
KAPPA = 0.99
THRESHOLD = 20

NP_ = 10240
NCORES = 2
NSUB = 16
NW = NCORES * NSUB
CH = 128
PW = 5120
EPAD = NW * PW
NCHUNK = PW // CH
ROWS_PER_TILE = NP_ // NSUB

_mesh = plsc.VectorSubcoreMesh(core_axis_name="c", subcore_axis_name="s")


@functools.partial(
    pl.kernel,
    mesh=_mesh,
    out_type=jax.ShapeDtypeStruct((NCORES, NP_, NHID), jnp.float32),
    scratch_types=[
        pltpu.VMEM((CH,), jnp.int32),
        pltpu.VMEM((CH,), jnp.int32),
        pltpu.VMEM((CH,), jnp.float32),
        pltpu.VMEM((CH, NHID), jnp.float32),
        pltpu.VMEM_SHARED((NP_, NHID), jnp.float32),
        pltpu.SemaphoreType.DMA,
    ],
)
def _sc_spmm(x_hbm, col_hbm, row_hbm, val_hbm, out_hbm,
             cidx, ridx, vals, rows, acc, sem):
    cid = lax.axis_index("c")
    sid = lax.axis_index("s")
    wid = cid * NSUB + sid

    def _zrow(i, _):
        for g in range(NHID // 16):
            rows[i, g * 16:(g + 1) * 16] = jnp.zeros((16,), jnp.float32)
        return 0
    lax.fori_loop(0, CH, _zrow, 0)
    r0 = sid * ROWS_PER_TILE
    for j in range(ROWS_PER_TILE // CH):
        pltpu.sync_copy(rows, acc.at[pl.ds(r0 + j * CH, CH)])
    plsc.subcore_barrier()

    def _chunk(j, _):
        eb = wid * PW + j * CH
        pltpu.sync_copy(col_hbm.at[pl.ds(eb, CH)], cidx)
        pltpu.sync_copy(row_hbm.at[pl.ds(eb, CH)], ridx)
        pltpu.sync_copy(val_hbm.at[pl.ds(eb, CH)], vals)
        pltpu.async_copy(x_hbm.at[cidx], rows, sem).wait()

        def _scale(b, _):
            vv16 = vals[pl.ds(b * 16, 16)]
            for lane in range(16):
                e = b * 16 + lane
                vb = jnp.full((16,), vv16[lane], jnp.float32)
                for g in range(NHID // 16):
                    sl = slice(g * 16, (g + 1) * 16)
                    rows[e, sl] = rows[e, sl] * vb
            return 0
        lax.fori_loop(0, CH // 16, _scale, 0)
        pltpu.sync_copy(rows, acc.at[ridx], add=True)
        return 0
    lax.fori_loop(0, NCHUNK, _chunk, 0)
    plsc.subcore_barrier()

    pltpu.sync_copy(acc.at[pl.ds(r0, ROWS_PER_TILE)],
                    out_hbm.at[cid, pl.ds(r0, ROWS_PER_TILE)])


@functools.partial(
    pl.kernel,
    mesh=_mesh,
    out_type=jax.ShapeDtypeStruct((NCORES, NP_), jnp.float32),
    compiler_params=pltpu.CompilerParams(needs_layout_passes=False),
    scratch_types=[
        pltpu.VMEM((CH,), jnp.int32),
        pltpu.VMEM((CH,), jnp.int32),
        pltpu.VMEM((CH,), jnp.float32),
        pltpu.VMEM((NP_,), jnp.float32),
        pltpu.VMEM((NP_,), jnp.float32),
        pltpu.VMEM((ROWS_PER_TILE,), jnp.float32),
        pltpu.VMEM((ROWS_PER_TILE,), jnp.float32),
        pltpu.VMEM_SHARED((NSUB, NP_), jnp.float32),
    ],
)
def _sc_matvec(v_hbm, col_hbm, row_hbm, val_hbm, out_hbm,
               cidx, ridx, vals, v_v, w_v, tmp, wsum, parts):
    cid = lax.axis_index("c")
    sid = lax.axis_index("s")
    wid = cid * NSUB + sid

    pltpu.sync_copy(v_hbm, v_v)

    def _zero(i, _):
        w_v[pl.ds(i * 16, 16)] = jnp.zeros((16,), jnp.float32)
        return 0
    lax.fori_loop(0, NP_ // 16, _zero, 0)

    def _chunk(j, _):
        eb = wid * PW + j * CH
        pltpu.sync_copy(col_hbm.at[pl.ds(eb, CH)], cidx)
        pltpu.sync_copy(row_hbm.at[pl.ds(eb, CH)], ridx)
        pltpu.sync_copy(val_hbm.at[pl.ds(eb, CH)], vals)
        for g in range(CH // 16):
            sl = pl.ds(g * 16, 16)
            gath = plsc.load_gather(v_v, [cidx[sl]])
            plsc.addupdate_scatter(w_v, [ridx[sl]], gath * vals[sl])
        return 0
    lax.fori_loop(0, NCHUNK, _chunk, 0)

    pltpu.sync_copy(w_v, parts.at[sid])
    plsc.subcore_barrier()

    r0 = sid * ROWS_PER_TILE
    def _acc_zero(i, _):
        wsum[pl.ds(i * 16, 16)] = jnp.zeros((16,), jnp.float32)
        return 0
    lax.fori_loop(0, ROWS_PER_TILE // 16, _acc_zero, 0)
    for p in range(NSUB):
        pltpu.sync_copy(parts.at[p, pl.ds(r0, ROWS_PER_TILE)], tmp)
        def _add(i, _):
            sl = pl.ds(i * 16, 16)
            wsum[sl] = wsum[sl] + tmp[sl]
            return 0
        lax.fori_loop(0, ROWS_PER_TILE // 16, _add, 0)
    pltpu.sync_copy(wsum, out_hbm.at[cid, pl.ds(r0, ROWS_PER_TILE)])


_RB = 1024


def _tc_mm(S, M, A=None, relu=False):
    P = S.shape[0]
    grid = NP_ // _RB

    def body(*refs):
        if A is not None:
            s_ref, m_ref, a_ref, o_ref = refs
        else:
            s_ref, m_ref, o_ref = refs
        acc = s_ref[0]
        for p in range(1, P):
            acc = acc + s_ref[p]
        r = jnp.dot(acc, m_ref[...], preferred_element_type=jnp.float32)
        if A is not None:
            r = r + a_ref[...]
        if relu:
            r = jnp.maximum(r, 0.0)
        o_ref[...] = r

    in_specs = [
        pl.BlockSpec((P, _RB, NHID), lambda i: (0, i, 0)),
        pl.BlockSpec((NHID, NHID), lambda i: (0, 0)),
    ]
    args = [S, M]
    if A is not None:
        in_specs.append(pl.BlockSpec((_RB, NHID), lambda i: (i, 0)))
        args.append(A)
    return pl.pallas_call(
        body,
        grid=(grid,),
        in_specs=in_specs,
        out_specs=pl.BlockSpec((_RB, NHID), lambda i: (i, 0)),
        out_shape=jax.ShapeDtypeStruct((NP_, NHID), jnp.float32),
    )(*args)


def _proj_norm_inf(Wm, kappa):
    sign = jnp.sign(Wm)
    a = jnp.abs(Wm)
    s = a.sum(axis=-1)
    u = jnp.sort(a, axis=-1)[:, ::-1]
    css = jnp.cumsum(u, axis=-1)
    j = jnp.arange(1, a.shape[-1] + 1, dtype=a.dtype)
    cond = u * j > (css - kappa)
    rho = jnp.maximum(jnp.sum(cond, axis=-1), 1)
    theta = (jnp.take_along_axis(css, (rho - 1)[:, None], axis=-1)[:, 0]
             - kappa) / rho.astype(a.dtype)
    proj = jnp.maximum(a - theta[:, None], 0.0)
    return jnp.where((s > kappa)[:, None], sign * proj, Wm)


def kernel(U, edge_index, A_values, W, B, V_w):
    row = edge_index[0]
    col = edge_index[1]
    epad = EPAD - E
    row_p = jnp.concatenate([row, jnp.zeros((epad,), row.dtype)])
    col_p = jnp.concatenate([col, jnp.zeros((epad,), col.dtype)])
    val_p = jnp.concatenate([A_values, jnp.zeros((epad,), A_values.dtype)])
    aval_p = jnp.abs(val_p)

    v = jnp.zeros((NP_,), jnp.float32).at[:N].set(1.0 / jnp.sqrt(jnp.float32(N)))
    for _ in range(50):
        wparts = _sc_matvec(v, col_p, row_p, aval_p)
        w = wparts[0] + wparts[1]
        v = w / (jnp.linalg.norm(w) + 1e-12)
    wparts = _sc_matvec(v, col_p, row_p, aval_p)
    w = wparts[0] + wparts[1]
    rho = jnp.linalg.norm(w) + 1e-5

    Wp = _proj_norm_inf(W, KAPPA / rho)

    U_p = jnp.zeros((NP_, NFEAT), jnp.float32).at[:N].set(U)
    AU = _sc_spmm(U_p, col_p, row_p, val_p)
    AUB = _tc_mm(AU, B)

    def f(z2d):
        S = _sc_spmm(z2d, col_p, row_p, val_p)
        return _tc_mm(S, Wp, A=AUB, relu=True)

    m, lam = 5, 1e-4
    X = [None] * m
    F = [None] * m
    X[0] = jnp.zeros((NP_ * NHID,), jnp.float32)
    F[0] = f(X[0].reshape(NP_, NHID)).reshape(-1)
    X[1] = F[0]
    F[1] = f(F[0].reshape(NP_, NHID)).reshape(-1)
    for k in range(2, THRESHOLD):
        nn_ = min(k, m)
        Xs = jnp.stack([X[i] for i in range(nn_)])
        Fs = jnp.stack([F[i] for i in range(nn_)])
        G = Fs - Xs
        H = jnp.zeros((nn_ + 1, nn_ + 1), jnp.float32)
        H = H.at[0, 1:].set(1.0).at[1:, 0].set(1.0)
        H = H.at[1:, 1:].set(G @ G.T + lam * jnp.eye(nn_, dtype=jnp.float32))
        y = jnp.zeros((nn_ + 1,), jnp.float32).at[0].set(1.0)
        alpha = jnp.linalg.solve(H, y)[1:]
        xk = alpha @ Fs
        X[k % m] = xk
        F[k % m] = f(xk.reshape(NP_, NHID)).reshape(-1)
    z_star = F[(THRESHOLD - 1) % m].reshape(NP_, NHID)

    VwT = jnp.zeros((NHID, NHID), jnp.float32).at[:, :NCLASS].set(V_w.T)
    label_pred = _tc_mm(z_star[None], VwT)[:N, :NCLASS]
    return (label_pred, z_star[:N])


# speedup vs baseline: 7.5223x; 1.1953x over previous
import functools
import jax
import jax.numpy as jnp
from jax import lax
from jax.experimental import pallas as pl
from jax.experimental.pallas import tpu as pltpu
from jax.experimental.pallas import tpu_sc as plsc

N = 10000
E = 160000
NFEAT = 128
NHID = 128
NCLASS = 16
KAPPA = 0.99
THRESHOLD = 20

NP_ = 10240
NCORES = 2
NSUB = 16
NW = NCORES * NSUB
CH = 128
PW = 5120
EPAD = NW * PW
NCHUNK = PW // CH
ROWS_PER_TILE = NP_ // NSUB

_mesh = plsc.VectorSubcoreMesh(core_axis_name="c", subcore_axis_name="s")


@functools.partial(
    pl.kernel,
    mesh=_mesh,
    out_type=jax.ShapeDtypeStruct((NCORES, NP_, NHID), jnp.float32),
    scratch_types=[
        pltpu.VMEM((CH,), jnp.int32),
        pltpu.VMEM((CH,), jnp.int32),
        pltpu.VMEM((CH,), jnp.float32),
        pltpu.VMEM((CH, NHID), jnp.float32),
        pltpu.VMEM_SHARED((NP_, NHID), jnp.float32),
        pltpu.SemaphoreType.DMA,
    ],
)
def _sc_spmm(x_hbm, col_hbm, row_hbm, val_hbm, out_hbm,
             cidx, ridx, vals, rows, acc, sem):
    cid = lax.axis_index("c")
    sid = lax.axis_index("s")
    wid = cid * NSUB + sid

    def _zrow(i, _):
        for g in range(NHID // 16):
            rows[i, g * 16:(g + 1) * 16] = jnp.zeros((16,), jnp.float32)
        return 0
    lax.fori_loop(0, CH, _zrow, 0)
    r0 = sid * ROWS_PER_TILE
    for j in range(ROWS_PER_TILE // CH):
        pltpu.sync_copy(rows, acc.at[pl.ds(r0 + j * CH, CH)])
    plsc.subcore_barrier()

    def _chunk(j, _):
        eb = wid * PW + j * CH
        pltpu.sync_copy(col_hbm.at[pl.ds(eb, CH)], cidx)
        pltpu.sync_copy(row_hbm.at[pl.ds(eb, CH)], ridx)
        pltpu.sync_copy(val_hbm.at[pl.ds(eb, CH)], vals)
        pltpu.async_copy(x_hbm.at[cidx], rows, sem).wait()

        def _scale(b, _):
            vv16 = vals[pl.ds(b * 16, 16)]
            for lane in range(16):
                e = b * 16 + lane
                vb = jnp.full((16,), vv16[lane], jnp.float32)
                for g in range(NHID // 16):
                    sl = slice(g * 16, (g + 1) * 16)
                    rows[e, sl] = rows[e, sl] * vb
            return 0
        lax.fori_loop(0, CH // 16, _scale, 0)
        pltpu.sync_copy(rows, acc.at[ridx], add=True)
        return 0
    lax.fori_loop(0, NCHUNK, _chunk, 0)
    plsc.subcore_barrier()

    pltpu.sync_copy(acc.at[pl.ds(r0, ROWS_PER_TILE)],
                    out_hbm.at[cid, pl.ds(r0, ROWS_PER_TILE)])


EPT = EPAD // NSUB


@functools.partial(
    pl.kernel,
    mesh=_mesh,
    out_type=jax.ShapeDtypeStruct((NP_,), jnp.float32),
    compiler_params=pltpu.CompilerParams(needs_layout_passes=False),
    scratch_types=[
        pltpu.VMEM((EPT,), jnp.int32),
        pltpu.VMEM((EPT,), jnp.int32),
        pltpu.VMEM((EPT,), jnp.float32),
        pltpu.VMEM((NP_,), jnp.float32),
        pltpu.VMEM((NP_,), jnp.float32),
        pltpu.VMEM((ROWS_PER_TILE,), jnp.float32),
        pltpu.VMEM((ROWS_PER_TILE,), jnp.float32),
        pltpu.VMEM((16,), jnp.float32),
        pltpu.VMEM((NSUB, 16), jnp.float32),
        pltpu.VMEM_SHARED((NSUB, NP_), jnp.float32),
        pltpu.VMEM_SHARED((NP_,), jnp.float32),
        pltpu.VMEM_SHARED((NSUB, 16), jnp.float32),
    ],
)
def _sc_power(v0_hbm, col_hbm, row_hbm, aval_hbm, out_hbm,
              cols, rowsi, avals, v_v, w_v, tmp, wsum, sq, sqall,
              parts, vsh, sqparts):
    cid = lax.axis_index("c")
    sid = lax.axis_index("s")

    @pl.when(cid == 0)
    def _run():
        eb = sid * EPT
        pltpu.sync_copy(col_hbm.at[pl.ds(eb, EPT)], cols)
        pltpu.sync_copy(row_hbm.at[pl.ds(eb, EPT)], rowsi)
        pltpu.sync_copy(aval_hbm.at[pl.ds(eb, EPT)], avals)
        pltpu.sync_copy(v0_hbm, v_v)
        r0 = sid * ROWS_PER_TILE

        def _matvec_reduce():
            def _zero(i, _):
                w_v[pl.ds(i * 16, 16)] = jnp.zeros((16,), jnp.float32)
                return 0
            lax.fori_loop(0, NP_ // 16, _zero, 0)

            def _edge(i, _):
                sl = pl.ds(i * 16, 16)
                g = plsc.load_gather(v_v, [cols[sl]])
                plsc.addupdate_scatter(w_v, [rowsi[sl]], g * avals[sl])
                return 0
            lax.fori_loop(0, EPT // 16, _edge, 0)

            pltpu.sync_copy(w_v, parts.at[sid])
            plsc.subcore_barrier()

            def _rzero(i, _):
                wsum[pl.ds(i * 16, 16)] = jnp.zeros((16,), jnp.float32)
                return 0
            lax.fori_loop(0, ROWS_PER_TILE // 16, _rzero, 0)
            for p in range(NSUB):
                pltpu.sync_copy(parts.at[p, pl.ds(r0, ROWS_PER_TILE)], tmp)

                def _radd(i, _):
                    sl = pl.ds(i * 16, 16)
                    wsum[sl] = wsum[sl] + tmp[sl]
                    return 0
                lax.fori_loop(0, ROWS_PER_TILE // 16, _radd, 0)

        def _iter(_, c):
            _matvec_reduce()
            def _ssq(i, a):
                x = wsum[pl.ds(i * 16, 16)]
                return a + x * x
            ssq = lax.fori_loop(0, ROWS_PER_TILE // 16,
                                _ssq, jnp.zeros((16,), jnp.float32))
            s = jnp.broadcast_to(jnp.sum(ssq, axis=0), (16,))
            sq[...] = s
            pltpu.sync_copy(sq, sqparts.at[sid])
            plsc.subcore_barrier()
            pltpu.sync_copy(sqparts, sqall)
            tot = jnp.zeros((16,), jnp.float32)
            for p in range(NSUB):
                tot = tot + sqall[p]
            ii = plsc.bitcast(tot, jnp.int32)
            y = plsc.bitcast(jnp.int32(0x5F3759DF) - (ii >> 1), jnp.float32)
            for _n in range(4):
                y = y * (1.5 - 0.5 * tot * y * y)

            def _scale(i, _):
                sl = pl.ds(i * 16, 16)
                wsum[sl] = wsum[sl] * y
                return 0
            lax.fori_loop(0, ROWS_PER_TILE // 16, _scale, 0)
            pltpu.sync_copy(wsum, vsh.at[pl.ds(r0, ROWS_PER_TILE)])
            plsc.subcore_barrier()
            pltpu.sync_copy(vsh, v_v)
            plsc.subcore_barrier()
            return c
        lax.fori_loop(0, 50, _iter, 0)

        _matvec_reduce()
        pltpu.sync_copy(wsum, out_hbm.at[pl.ds(r0, ROWS_PER_TILE)])


_RB = 1024


def _tc_mm(S, M, A=None, relu=False):
    P = S.shape[0]
    grid = NP_ // _RB

    def body(*refs):
        if A is not None:
            s_ref, m_ref, a_ref, o_ref = refs
        else:
            s_ref, m_ref, o_ref = refs
        acc = s_ref[0]
        for p in range(1, P):
            acc = acc + s_ref[p]
        r = jnp.dot(acc, m_ref[...], preferred_element_type=jnp.float32)
        if A is not None:
            r = r + a_ref[...]
        if relu:
            r = jnp.maximum(r, 0.0)
        o_ref[...] = r

    in_specs = [
        pl.BlockSpec((P, _RB, NHID), lambda i: (0, i, 0)),
        pl.BlockSpec((NHID, NHID), lambda i: (0, 0)),
    ]
    args = [S, M]
    if A is not None:
        in_specs.append(pl.BlockSpec((_RB, NHID), lambda i: (i, 0)))
        args.append(A)
    return pl.pallas_call(
        body,
        grid=(grid,),
        in_specs=in_specs,
        out_specs=pl.BlockSpec((_RB, NHID), lambda i: (i, 0)),
        out_shape=jax.ShapeDtypeStruct((NP_, NHID), jnp.float32),
    )(*args)


def _proj_norm_inf(Wm, kappa):
    sign = jnp.sign(Wm)
    a = jnp.abs(Wm)
    s = a.sum(axis=-1)
    u = jnp.sort(a, axis=-1)[:, ::-1]
    css = jnp.cumsum(u, axis=-1)
    j = jnp.arange(1, a.shape[-1] + 1, dtype=a.dtype)
    cond = u * j > (css - kappa)
    rho = jnp.maximum(jnp.sum(cond, axis=-1), 1)
    theta = (jnp.take_along_axis(css, (rho - 1)[:, None], axis=-1)[:, 0]
             - kappa) / rho.astype(a.dtype)
    proj = jnp.maximum(a - theta[:, None], 0.0)
    return jnp.where((s > kappa)[:, None], sign * proj, Wm)


def kernel(U, edge_index, A_values, W, B, V_w):
    row = edge_index[0]
    col = edge_index[1]
    epad = EPAD - E
    row_p = jnp.concatenate([row, jnp.zeros((epad,), row.dtype)])
    col_p = jnp.concatenate([col, jnp.zeros((epad,), col.dtype)])
    val_p = jnp.concatenate([A_values, jnp.zeros((epad,), A_values.dtype)])
    aval_p = jnp.abs(val_p)

    v0 = jnp.zeros((NP_,), jnp.float32).at[:N].set(1.0 / jnp.sqrt(jnp.float32(N)))
    w = _sc_power(v0, col_p, row_p, aval_p)
    rho = jnp.linalg.norm(w) + 1e-5

    Wp = _proj_norm_inf(W, KAPPA / rho)

    U_p = jnp.zeros((NP_, NFEAT), jnp.float32).at[:N].set(U)
    AU = _sc_spmm(U_p, col_p, row_p, val_p)
    AUB = _tc_mm(AU, B)

    def f(z2d):
        S = _sc_spmm(z2d, col_p, row_p, val_p)
        return _tc_mm(S, Wp, A=AUB, relu=True)

    m, lam = 5, 1e-4
    X = [None] * m
    F = [None] * m
    X[0] = jnp.zeros((NP_ * NHID,), jnp.float32)
    F[0] = f(X[0].reshape(NP_, NHID)).reshape(-1)
    X[1] = F[0]
    F[1] = f(F[0].reshape(NP_, NHID)).reshape(-1)
    for k in range(2, THRESHOLD):
        nn_ = min(k, m)
        Xs = jnp.stack([X[i] for i in range(nn_)])
        Fs = jnp.stack([F[i] for i in range(nn_)])
        G = Fs - Xs
        H = jnp.zeros((nn_ + 1, nn_ + 1), jnp.float32)
        H = H.at[0, 1:].set(1.0).at[1:, 0].set(1.0)
        H = H.at[1:, 1:].set(G @ G.T + lam * jnp.eye(nn_, dtype=jnp.float32))
        y = jnp.zeros((nn_ + 1,), jnp.float32).at[0].set(1.0)
        alpha = jnp.linalg.solve(H, y)[1:]
        xk = alpha @ Fs
        X[k % m] = xk
        F[k % m] = f(xk.reshape(NP_, NHID)).reshape(-1)
    z_star = F[(THRESHOLD - 1) % m].reshape(NP_, NHID)

    VwT = jnp.zeros((NHID, NHID), jnp.float32).at[:, :NCLASS].set(V_w.T)
    label_pred = _tc_mm(z_star[None], VwT)[:N, :NCLASS]
    return (label_pred, z_star[:N])


# speedup vs baseline: 9.0095x; 1.1977x over previous
import functools
import jax
import jax.numpy as jnp
from jax import lax
from jax.experimental import pallas as pl
from jax.experimental.pallas import tpu as pltpu
from jax.experimental.pallas import tpu_sc as plsc

N = 10000
E = 160000
NFEAT = 128
NHID = 128
NCLASS = 16
KAPPA = 0.99
THRESHOLD = 20

NP_ = 10240
NCORES = 2
NSUB = 16
NW = NCORES * NSUB
CH = 128
PW = 5120
EPAD = NW * PW
NCHUNK = PW // CH
ROWS_PER_TILE = NP_ // NSUB

_mesh = plsc.VectorSubcoreMesh(core_axis_name="c", subcore_axis_name="s")


@functools.partial(
    pl.kernel,
    mesh=_mesh,
    out_type=jax.ShapeDtypeStruct((NCORES, NP_, NHID), jnp.float32),
    scratch_types=[
        pltpu.VMEM((NCHUNK, CH), jnp.int32),
        pltpu.VMEM((NCHUNK, CH), jnp.int32),
        pltpu.VMEM((NCHUNK, CH), jnp.float32),
        pltpu.VMEM((CH, NHID), jnp.float32),
        pltpu.VMEM((CH, NHID), jnp.float32),
        pltpu.VMEM((CH,), jnp.int32),
        pltpu.VMEM((CH,), jnp.int32),
        pltpu.VMEM_SHARED((NP_, NHID), jnp.float32),
        pltpu.SemaphoreType.DMA,
        pltpu.SemaphoreType.DMA,
        pltpu.SemaphoreType.DMA,
        pltpu.SemaphoreType.DMA,
    ],
)
def _sc_spmm(x_hbm, col_hbm, row_hbm, val_hbm, out_hbm,
             cidx, ridx, vals, rows0, rows1, rix0, rix1, acc,
             sg0, sg1, ss0, ss1):
    cid = lax.axis_index("c")
    sid = lax.axis_index("s")
    wid = cid * NSUB + sid
    bufs = ((rows0, rix0, sg0, ss0), (rows1, rix1, sg1, ss1))

    pltpu.sync_copy(col_hbm.at[wid], cidx)
    pltpu.sync_copy(row_hbm.at[wid], ridx)
    pltpu.sync_copy(val_hbm.at[wid], vals)

    def _zrow(i, _):
        for g in range(NHID // 16):
            rows0[i, g * 16:(g + 1) * 16] = jnp.zeros((16,), jnp.float32)
        return 0
    lax.fori_loop(0, CH, _zrow, 0)
    r0 = sid * ROWS_PER_TILE
    for j in range(ROWS_PER_TILE // CH):
        pltpu.sync_copy(rows0, acc.at[pl.ds(r0 + j * CH, CH)])
    plsc.subcore_barrier()

    pltpu.async_copy(x_hbm.at[cidx.at[0]], rows0, sg0)

    def _pair(j2, _):
        for b in range(2):
            j = 2 * j2 + b
            rbuf, rix, sg, ss = bufs[b]
            nbuf, nrix, nsg, nss = bufs[1 - b]

            @pl.when(j >= 1)
            def _drain_prev():
                pltpu.make_async_copy(nbuf, acc.at[nrix], nss).wait()

            @pl.when(j + 1 < NCHUNK)
            def _next_gather():
                pltpu.async_copy(x_hbm.at[cidx.at[j + 1]], nbuf, nsg)

            pltpu.make_async_copy(x_hbm.at[cidx.at[j]], rbuf, sg).wait()

            for i8 in range(CH // 16):
                sli = pl.ds(i8 * 16, 16)
                rix[sli] = ridx[j, sli]

            def _scale(g16, _):
                vv16 = vals[j, pl.ds(g16 * 16, 16)]
                for lane in range(16):
                    e = g16 * 16 + lane
                    vb = jnp.full((16,), vv16[lane], jnp.float32)
                    for g in range(NHID // 16):
                        sl = slice(g * 16, (g + 1) * 16)
                        rbuf[e, sl] = rbuf[e, sl] * vb
                return 0
            lax.fori_loop(0, CH // 16, _scale, 0)
            pltpu.async_copy(rbuf, acc.at[rix], ss, add=True)
        return 0
    lax.fori_loop(0, NCHUNK // 2, _pair, 0)
    pltpu.make_async_copy(rows1, acc.at[rix1], ss1).wait()
    plsc.subcore_barrier()

    pltpu.sync_copy(acc.at[pl.ds(r0, ROWS_PER_TILE)],
                    out_hbm.at[cid, pl.ds(r0, ROWS_PER_TILE)])


EPT = EPAD // NSUB


@functools.partial(
    pl.kernel,
    mesh=_mesh,
    out_type=jax.ShapeDtypeStruct((NP_,), jnp.float32),
    compiler_params=pltpu.CompilerParams(needs_layout_passes=False),
    scratch_types=[
        pltpu.VMEM((EPT,), jnp.int32),
        pltpu.VMEM((EPT,), jnp.int32),
        pltpu.VMEM((EPT,), jnp.float32),
        pltpu.VMEM((NP_,), jnp.float32),
        pltpu.VMEM((NP_,), jnp.float32),
        pltpu.VMEM((ROWS_PER_TILE,), jnp.float32),
        pltpu.VMEM((ROWS_PER_TILE,), jnp.float32),
        pltpu.VMEM((16,), jnp.float32),
        pltpu.VMEM((NSUB, 16), jnp.float32),
        pltpu.VMEM_SHARED((NSUB, NP_), jnp.float32),
        pltpu.VMEM_SHARED((NP_,), jnp.float32),
        pltpu.VMEM_SHARED((NSUB, 16), jnp.float32),
    ],
)
def _sc_power(v0_hbm, col_hbm, row_hbm, aval_hbm, out_hbm,
              cols, rowsi, avals, v_v, w_v, tmp, wsum, sq, sqall,
              parts, vsh, sqparts):
    cid = lax.axis_index("c")
    sid = lax.axis_index("s")

    @pl.when(cid == 0)
    def _run():
        eb = sid * EPT
        pltpu.sync_copy(col_hbm.at[pl.ds(eb, EPT)], cols)
        pltpu.sync_copy(row_hbm.at[pl.ds(eb, EPT)], rowsi)
        pltpu.sync_copy(aval_hbm.at[pl.ds(eb, EPT)], avals)
        pltpu.sync_copy(v0_hbm, v_v)
        r0 = sid * ROWS_PER_TILE

        def _matvec_reduce():
            def _zero(i, _):
                w_v[pl.ds(i * 16, 16)] = jnp.zeros((16,), jnp.float32)
                return 0
            lax.fori_loop(0, NP_ // 16, _zero, 0)

            def _edge(i, _):
                sl = pl.ds(i * 16, 16)
                g = plsc.load_gather(v_v, [cols[sl]])
                plsc.addupdate_scatter(w_v, [rowsi[sl]], g * avals[sl])
                return 0
            lax.fori_loop(0, EPT // 16, _edge, 0)

            pltpu.sync_copy(w_v, parts.at[sid])
            plsc.subcore_barrier()

            def _rzero(i, _):
                wsum[pl.ds(i * 16, 16)] = jnp.zeros((16,), jnp.float32)
                return 0
            lax.fori_loop(0, ROWS_PER_TILE // 16, _rzero, 0)
            for p in range(NSUB):
                pltpu.sync_copy(parts.at[p, pl.ds(r0, ROWS_PER_TILE)], tmp)

                def _radd(i, _):
                    sl = pl.ds(i * 16, 16)
                    wsum[sl] = wsum[sl] + tmp[sl]
                    return 0
                lax.fori_loop(0, ROWS_PER_TILE // 16, _radd, 0)

        def _iter(_, c):
            _matvec_reduce()
            def _ssq(i, a):
                x = wsum[pl.ds(i * 16, 16)]
                return a + x * x
            ssq = lax.fori_loop(0, ROWS_PER_TILE // 16,
                                _ssq, jnp.zeros((16,), jnp.float32))
            s = jnp.broadcast_to(jnp.sum(ssq, axis=0), (16,))
            sq[...] = s
            pltpu.sync_copy(sq, sqparts.at[sid])
            plsc.subcore_barrier()
            pltpu.sync_copy(sqparts, sqall)
            tot = jnp.zeros((16,), jnp.float32)
            for p in range(NSUB):
                tot = tot + sqall[p]
            ii = plsc.bitcast(tot, jnp.int32)
            y = plsc.bitcast(jnp.int32(0x5F3759DF) - (ii >> 1), jnp.float32)
            for _n in range(4):
                y = y * (1.5 - 0.5 * tot * y * y)

            def _scale(i, _):
                sl = pl.ds(i * 16, 16)
                wsum[sl] = wsum[sl] * y
                return 0
            lax.fori_loop(0, ROWS_PER_TILE // 16, _scale, 0)
            pltpu.sync_copy(wsum, vsh.at[pl.ds(r0, ROWS_PER_TILE)])
            plsc.subcore_barrier()
            pltpu.sync_copy(vsh, v_v)
            plsc.subcore_barrier()
            return c
        lax.fori_loop(0, 50, _iter, 0)

        _matvec_reduce()
        pltpu.sync_copy(wsum, out_hbm.at[pl.ds(r0, ROWS_PER_TILE)])


_RB = 1024


def _tc_mm(S, M, A=None, relu=False):
    P = S.shape[0]
    grid = NP_ // _RB

    def body(*refs):
        if A is not None:
            s_ref, m_ref, a_ref, o_ref = refs
        else:
            s_ref, m_ref, o_ref = refs
        acc = s_ref[0]
        for p in range(1, P):
            acc = acc + s_ref[p]
        r = jnp.dot(acc, m_ref[...], preferred_element_type=jnp.float32)
        if A is not None:
            r = r + a_ref[...]
        if relu:
            r = jnp.maximum(r, 0.0)
        o_ref[...] = r

    in_specs = [
        pl.BlockSpec((P, _RB, NHID), lambda i: (0, i, 0)),
        pl.BlockSpec((NHID, NHID), lambda i: (0, 0)),
    ]
    args = [S, M]
    if A is not None:
        in_specs.append(pl.BlockSpec((_RB, NHID), lambda i: (i, 0)))
        args.append(A)
    return pl.pallas_call(
        body,
        grid=(grid,),
        in_specs=in_specs,
        out_specs=pl.BlockSpec((_RB, NHID), lambda i: (i, 0)),
        out_shape=jax.ShapeDtypeStruct((NP_, NHID), jnp.float32),
    )(*args)


def _proj_norm_inf(Wm, kappa):
    sign = jnp.sign(Wm)
    a = jnp.abs(Wm)
    s = a.sum(axis=-1)
    u = jnp.sort(a, axis=-1)[:, ::-1]
    css = jnp.cumsum(u, axis=-1)
    j = jnp.arange(1, a.shape[-1] + 1, dtype=a.dtype)
    cond = u * j > (css - kappa)
    rho = jnp.maximum(jnp.sum(cond, axis=-1), 1)
    theta = (jnp.take_along_axis(css, (rho - 1)[:, None], axis=-1)[:, 0]
             - kappa) / rho.astype(a.dtype)
    proj = jnp.maximum(a - theta[:, None], 0.0)
    return jnp.where((s > kappa)[:, None], sign * proj, Wm)


def kernel(U, edge_index, A_values, W, B, V_w):
    row = edge_index[0]
    col = edge_index[1]
    epad = EPAD - E
    row_p = jnp.concatenate([row, jnp.zeros((epad,), row.dtype)])
    col_p = jnp.concatenate([col, jnp.zeros((epad,), col.dtype)])
    val_p = jnp.concatenate([A_values, jnp.zeros((epad,), A_values.dtype)])
    aval_p = jnp.abs(val_p)
    col3 = col_p.reshape(NW, NCHUNK, CH)
    row3 = row_p.reshape(NW, NCHUNK, CH)
    val3 = val_p.reshape(NW, NCHUNK, CH)

    v0 = jnp.zeros((NP_,), jnp.float32).at[:N].set(1.0 / jnp.sqrt(jnp.float32(N)))
    w = _sc_power(v0, col_p, row_p, aval_p)
    rho = jnp.linalg.norm(w) + 1e-5

    Wp = _proj_norm_inf(W, KAPPA / rho)

    U_p = jnp.zeros((NP_, NFEAT), jnp.float32).at[:N].set(U)
    AU = _sc_spmm(U_p, col3, row3, val3)
    AUB = _tc_mm(AU, B)

    def f(z2d):
        S = _sc_spmm(z2d, col3, row3, val3)
        return _tc_mm(S, Wp, A=AUB, relu=True)

    m, lam = 5, 1e-4
    X = [None] * m
    F = [None] * m
    X[0] = jnp.zeros((NP_ * NHID,), jnp.float32)
    F[0] = f(X[0].reshape(NP_, NHID)).reshape(-1)
    X[1] = F[0]
    F[1] = f(F[0].reshape(NP_, NHID)).reshape(-1)
    for k in range(2, THRESHOLD):
        nn_ = min(k, m)
        Xs = jnp.stack([X[i] for i in range(nn_)])
        Fs = jnp.stack([F[i] for i in range(nn_)])
        G = Fs - Xs
        H = jnp.zeros((nn_ + 1, nn_ + 1), jnp.float32)
        H = H.at[0, 1:].set(1.0).at[1:, 0].set(1.0)
        H = H.at[1:, 1:].set(G @ G.T + lam * jnp.eye(nn_, dtype=jnp.float32))
        y = jnp.zeros((nn_ + 1,), jnp.float32).at[0].set(1.0)
        alpha = jnp.linalg.solve(H, y)[1:]
        xk = alpha @ Fs
        X[k % m] = xk
        F[k % m] = f(xk.reshape(NP_, NHID)).reshape(-1)
    z_star = F[(THRESHOLD - 1) % m].reshape(NP_, NHID)

    VwT = jnp.zeros((NHID, NHID), jnp.float32).at[:, :NCLASS].set(V_w.T)
    label_pred = _tc_mm(z_star[None], VwT)[:N, :NCLASS]
    return (label_pred, z_star[:N])
